# Initial kernel scaffold; baseline (speedup 1.0000x reference)
import jax
import jax.numpy as jnp
from jax.experimental import pallas as pl


def kernel(y_pred):
    raise NotImplementedError("write your pallas kernel here")


# speedup vs baseline: 8.4892x; 8.4892x over previous
import jax
import jax.numpy as jnp
from jax.experimental import pallas as pl

INPUT_H, INPUT_W = 300, 300
NMS_MAX = 100
CONF_TH = 0.01
IOU_TH = 0.45
NUM_PRED = 100
N_CLASSES = 21
NEG = -1e9
_B = 4
_N = 5000
_NP = 5120
_C = N_CLASSES - 1
_R = _B * _C
_FLAT = _C * NMS_MAX
_FLATP = 2048


def _nms_body(sc_ref, box_ref, out_ref):
    off0 = box_ref[0]
    off1 = box_ref[1]
    off2 = box_ref[2]
    off3 = box_ref[3]
    anc0 = box_ref[4]
    anc1 = box_ref[5]
    anc2 = box_ref[6]
    anc3 = box_ref[7]
    var0 = box_ref[8]
    var1 = box_ref[9]
    var2 = box_ref[10]
    var3 = box_ref[11]
    cx = off0 * anc2 * var0 + anc0
    cy = off1 * anc3 * var1 + anc1
    w = anc2 * jnp.exp(off2 * var2)
    h = anc3 * jnp.exp(off3 * var3)
    xs = (cx - 0.5 * w) * INPUT_W
    ys = (cy - 0.5 * h) * INPUT_H
    xe = (cx + 0.5 * w) * INPUT_W
    ye = (cy + 0.5 * h) * INPUT_H

    def rep(a):
        return jnp.reshape(jnp.broadcast_to(a[:, None, :], (_B, _C, _NP)), (_R, _NP))

    xs = rep(xs)
    ys = rep(ys)
    xe = rep(xe)
    ye = rep(ye)
    area2 = jnp.maximum(xe - xs, 0.0) * jnp.maximum(ye - ys, 0.0)

    iota = jax.lax.broadcasted_iota(jnp.int32, (_R, _NP), 1)
    sc0 = sc_ref[...]
    sc_init = jnp.where(sc0 > CONF_TH, sc0, NEG)

    def pick(onehot, a):
        return jnp.sum(jnp.where(onehot, a, 0.0), axis=1, keepdims=True)

    def step(t, sc):
        m = jnp.max(sc, axis=1, keepdims=True)
        idx = jnp.min(jnp.where(sc == m, iota, _NP), axis=1, keepdims=True)
        onehot = iota == idx
        bx0 = pick(onehot, xs)
        by0 = pick(onehot, ys)
        bx1 = pick(onehot, xe)
        by1 = pick(onehot, ye)
        area1 = jnp.maximum(bx1 - bx0, 0.0) * jnp.maximum(by1 - by0, 0.0)
        iw = jnp.maximum(jnp.minimum(bx1, xe) - jnp.maximum(bx0, xs), 0.0)
        ih = jnp.maximum(jnp.minimum(by1, ye) - jnp.maximum(by0, ys), 0.0)
        inter = iw * ih
        union = area1 + area2 - inter
        iou = inter / jnp.maximum(union, 1e-8)
        suppress = (iou >= IOU_TH) | onehot
        valid = m > CONF_TH
        sc = jnp.where(suppress & valid, NEG, sc)
        vrow = valid[:, 0]
        out_ref[0, t, :] = jnp.where(vrow, m[:, 0], 0.0)
        out_ref[1, t, :] = jnp.where(vrow, bx0[:, 0], 0.0)
        out_ref[2, t, :] = jnp.where(vrow, by0[:, 0], 0.0)
        out_ref[3, t, :] = jnp.where(vrow, bx1[:, 0], 0.0)
        out_ref[4, t, :] = jnp.where(vrow, by1[:, 0], 0.0)
        return sc

    jax.lax.fori_loop(0, NMS_MAX, step, sc_init)


def _topk_body(sc_ref, f_ref, out_ref):
    iota = jax.lax.broadcasted_iota(jnp.int32, (_B, _FLATP), 1)
    cls = (iota // NMS_MAX + 1).astype(jnp.float32)
    f0 = f_ref[0]
    f1 = f_ref[1]
    f2 = f_ref[2]
    f3 = f_ref[3]

    def pick(onehot, a):
        return jnp.sum(jnp.where(onehot, a, 0.0), axis=1)

    def step(k, sc):
        m = jnp.max(sc, axis=1, keepdims=True)
        idx = jnp.min(jnp.where(sc == m, iota, _FLATP), axis=1, keepdims=True)
        onehot = iota == idx
        out_ref[0, k, :] = pick(onehot, cls)
        out_ref[1, k, :] = m[:, 0]
        out_ref[2, k, :] = pick(onehot, f0)
        out_ref[3, k, :] = pick(onehot, f1)
        out_ref[4, k, :] = pick(onehot, f2)
        out_ref[5, k, :] = pick(onehot, f3)
        return jnp.where(onehot, NEG, sc)

    jax.lax.fori_loop(0, NUM_PRED, step, sc_ref[...])


@jax.jit
def kernel(y_pred):
    yt = jnp.transpose(y_pred, (2, 0, 1))
    yt = jnp.pad(yt, ((0, 0), (0, 0), (0, _NP - _N)))
    sc_in = jnp.reshape(yt[1:N_CLASSES].transpose(1, 0, 2), (_R, _NP))
    box_in = yt[N_CLASSES:]

    sel = pl.pallas_call(
        _nms_body,
        out_shape=jax.ShapeDtypeStruct((5, NMS_MAX, _R), jnp.float32),
    )(sc_in, box_in)

    flat = jnp.reshape(
        jnp.transpose(jnp.reshape(sel, (5, NMS_MAX, _B, _C)), (0, 2, 3, 1)),
        (5, _B, _FLAT),
    )
    sc_flat = jnp.pad(flat[0], ((0, 0), (0, _FLATP - _FLAT)), constant_values=NEG)
    f_flat = jnp.pad(flat[1:], ((0, 0), (0, 0), (0, _FLATP - _FLAT)))

    top = pl.pallas_call(
        _topk_body,
        out_shape=jax.ShapeDtypeStruct((6, NUM_PRED, _B), jnp.float32),
    )(sc_flat, f_flat)

    return jnp.transpose(top, (2, 1, 0))


# speedup vs baseline: 13.0555x; 1.5379x over previous
import jax
import jax.numpy as jnp
from jax.experimental import pallas as pl
from jax.experimental.pallas import tpu as pltpu

INPUT_H, INPUT_W = 300, 300
NMS_MAX = 100
CONF_TH = 0.01
IOU_TH = 0.45
NUM_PRED = 100
N_CLASSES = 21
NEG = -1e9
_B = 4
_N = 5000
_NP = 5120
_C = N_CLASSES - 1
_R = _B * _C
_FLAT = _C * NMS_MAX
_FLATP = 2048
_CK = 128
_NCK = _NP // _CK
_BIGI = 1e9


def _nms_body(sc_in, box_ref, out_ref, sc_s, xs_s, ys_s, xe_s, ye_s, a2_s, io_s):
    off0 = box_ref[0]
    off1 = box_ref[1]
    off2 = box_ref[2]
    off3 = box_ref[3]
    anc0 = box_ref[4]
    anc1 = box_ref[5]
    anc2 = box_ref[6]
    anc3 = box_ref[7]
    var0 = box_ref[8]
    var1 = box_ref[9]
    var2 = box_ref[10]
    var3 = box_ref[11]
    cx = off0 * anc2 * var0 + anc0
    cy = off1 * anc3 * var1 + anc1
    w = anc2 * jnp.exp(off2 * var2)
    h = anc3 * jnp.exp(off3 * var3)
    xs = (cx - 0.5 * w) * INPUT_W
    ys = (cy - 0.5 * h) * INPUT_H
    xe = (cx + 0.5 * w) * INPUT_W
    ye = (cy + 0.5 * h) * INPUT_H

    def rep(a):
        return jnp.reshape(jnp.broadcast_to(a[:, None, :], (_B, _C, _NP)), (_R, _NP))

    xs = rep(xs)
    ys = rep(ys)
    xe = rep(xe)
    ye = rep(ye)
    xs_s[...] = xs
    ys_s[...] = ys
    xe_s[...] = xe
    ye_s[...] = ye
    a2_s[...] = jnp.maximum(xe - xs, 0.0) * jnp.maximum(ye - ys, 0.0)
    io_s[...] = jax.lax.broadcasted_iota(jnp.int32, (_R, _NP), 1).astype(jnp.float32)

    sc0 = sc_in[...]
    sc_init = jnp.where(sc0 > CONF_TH, sc0, NEG)
    sc_s[...] = sc_init

    macc0 = sc_init[:, 0:_CK]
    for c in range(1, _NCK):
        macc0 = jnp.maximum(macc0, sc_init[:, c * _CK : (c + 1) * _CK])

    def step(t, macc):
        m = jnp.max(macc, axis=1, keepdims=True)
        iacc = None
        for c in range(_NCK):
            s = slice(c * _CK, (c + 1) * _CK)
            cand = jnp.where(sc_s[:, s] == m, io_s[:, s], _BIGI)
            iacc = jnp.minimum(iacc, cand) if c else cand
        idx = jnp.min(iacc, axis=1, keepdims=True)
        px0 = py0 = px1 = py1 = None
        for c in range(_NCK):
            s = slice(c * _CK, (c + 1) * _CK)
            oh = io_s[:, s] == idx
            if c == 0:
                px0 = jnp.where(oh, xs_s[:, s], 0.0)
                py0 = jnp.where(oh, ys_s[:, s], 0.0)
                px1 = jnp.where(oh, xe_s[:, s], 0.0)
                py1 = jnp.where(oh, ye_s[:, s], 0.0)
            else:
                px0 = jnp.where(oh, xs_s[:, s], px0)
                py0 = jnp.where(oh, ys_s[:, s], py0)
                px1 = jnp.where(oh, xe_s[:, s], px1)
                py1 = jnp.where(oh, ye_s[:, s], py1)
        bx0 = jnp.sum(px0, axis=1, keepdims=True)
        by0 = jnp.sum(py0, axis=1, keepdims=True)
        bx1 = jnp.sum(px1, axis=1, keepdims=True)
        by1 = jnp.sum(py1, axis=1, keepdims=True)
        area1 = jnp.maximum(bx1 - bx0, 0.0) * jnp.maximum(by1 - by0, 0.0)
        valid = m > CONF_TH
        macc_n = None
        for c in range(_NCK):
            s = slice(c * _CK, (c + 1) * _CK)
            iw = jnp.maximum(
                jnp.minimum(bx1, xe_s[:, s]) - jnp.maximum(bx0, xs_s[:, s]), 0.0
            )
            ih = jnp.maximum(
                jnp.minimum(by1, ye_s[:, s]) - jnp.maximum(by0, ys_s[:, s]), 0.0
            )
            inter = iw * ih
            union = area1 + a2_s[:, s] - inter
            iou = inter / jnp.maximum(union, 1e-8)
            sc_new = jnp.where(iou >= IOU_TH, NEG, sc_s[:, s])
            sc_s[:, s] = sc_new
            macc_n = sc_new if c == 0 else jnp.maximum(macc_n, sc_new)
        vrow = valid[:, 0]
        out_ref[0, t, :] = jnp.where(vrow, m[:, 0], 0.0)
        out_ref[1, t, :] = jnp.where(vrow, bx0[:, 0], 0.0)
        out_ref[2, t, :] = jnp.where(vrow, by0[:, 0], 0.0)
        out_ref[3, t, :] = jnp.where(vrow, bx1[:, 0], 0.0)
        out_ref[4, t, :] = jnp.where(vrow, by1[:, 0], 0.0)
        return macc_n

    jax.lax.fori_loop(0, NMS_MAX, step, macc0)


def _topk_body(sc_ref, f_ref, out_ref):
    iota = jax.lax.broadcasted_iota(jnp.int32, (_B, _FLATP), 1)
    cls = (iota // NMS_MAX + 1).astype(jnp.float32)
    f0 = f_ref[0]
    f1 = f_ref[1]
    f2 = f_ref[2]
    f3 = f_ref[3]

    def pick(onehot, a):
        return jnp.sum(jnp.where(onehot, a, 0.0), axis=1)

    def step(k, sc):
        m = jnp.max(sc, axis=1, keepdims=True)
        idx = jnp.min(jnp.where(sc == m, iota, _FLATP), axis=1, keepdims=True)
        onehot = iota == idx
        out_ref[0, k, :] = pick(onehot, cls)
        out_ref[1, k, :] = m[:, 0]
        out_ref[2, k, :] = pick(onehot, f0)
        out_ref[3, k, :] = pick(onehot, f1)
        out_ref[4, k, :] = pick(onehot, f2)
        out_ref[5, k, :] = pick(onehot, f3)
        return jnp.where(onehot, NEG, sc)

    jax.lax.fori_loop(0, NUM_PRED, step, sc_ref[...])


@jax.jit
def kernel(y_pred):
    yt = jnp.transpose(y_pred, (2, 0, 1))
    yt = jnp.pad(yt, ((0, 0), (0, 0), (0, _NP - _N)))
    sc_in = jnp.reshape(yt[1:N_CLASSES].transpose(1, 0, 2), (_R, _NP))
    box_in = yt[N_CLASSES:]

    sel = pl.pallas_call(
        _nms_body,
        out_shape=jax.ShapeDtypeStruct((5, NMS_MAX, _R), jnp.float32),
        scratch_shapes=[pltpu.VMEM((_R, _NP), jnp.float32)] * 7,
    )(sc_in, box_in)

    flat = jnp.reshape(
        jnp.transpose(jnp.reshape(sel, (5, NMS_MAX, _B, _C)), (0, 2, 3, 1)),
        (5, _B, _FLAT),
    )
    sc_flat = jnp.pad(flat[0], ((0, 0), (0, _FLATP - _FLAT)), constant_values=NEG)
    f_flat = jnp.pad(flat[1:], ((0, 0), (0, 0), (0, _FLATP - _FLAT)))

    top = pl.pallas_call(
        _topk_body,
        out_shape=jax.ShapeDtypeStruct((6, NUM_PRED, _B), jnp.float32),
    )(sc_flat, f_flat)

    return jnp.transpose(top, (2, 1, 0))


# speedup vs baseline: 13.0709x; 1.0012x over previous
import jax
import jax.numpy as jnp
from jax.experimental import pallas as pl
from jax.experimental.pallas import tpu as pltpu

INPUT_H, INPUT_W = 300, 300
NMS_MAX = 100
CONF_TH = 0.01
IOU_TH = 0.45
NUM_PRED = 100
N_CLASSES = 21
NEG = -1e9
_B = 4
_N = 5000
_NP = 5120
_C = N_CLASSES - 1
_R = _B * _C
_FLAT = _C * NMS_MAX
_FLATP = 2048
_CK = 128
_NCK = _NP // _CK
_BIGI = 1e9


def _nms_body(sc_in, box_ref, out_ref, sc_s, xs_s, ys_s, xe_s, ye_s, a2_s, io_s):
    off0 = box_ref[0]
    off1 = box_ref[1]
    off2 = box_ref[2]
    off3 = box_ref[3]
    anc0 = box_ref[4]
    anc1 = box_ref[5]
    anc2 = box_ref[6]
    anc3 = box_ref[7]
    var0 = box_ref[8]
    var1 = box_ref[9]
    var2 = box_ref[10]
    var3 = box_ref[11]
    cx = off0 * anc2 * var0 + anc0
    cy = off1 * anc3 * var1 + anc1
    w = anc2 * jnp.exp(off2 * var2)
    h = anc3 * jnp.exp(off3 * var3)
    xs = (cx - 0.5 * w) * INPUT_W
    ys = (cy - 0.5 * h) * INPUT_H
    xe = (cx + 0.5 * w) * INPUT_W
    ye = (cy + 0.5 * h) * INPUT_H

    def rep(a):
        return jnp.reshape(jnp.broadcast_to(a[:, None, :], (_B, _C, _NP)), (_R, _NP))

    xs = rep(xs)
    ys = rep(ys)
    xe = rep(xe)
    ye = rep(ye)
    xs_s[...] = xs
    ys_s[...] = ys
    xe_s[...] = xe
    ye_s[...] = ye
    a2_s[...] = jnp.maximum(xe - xs, 0.0) * jnp.maximum(ye - ys, 0.0)
    io_s[...] = jax.lax.broadcasted_iota(jnp.int32, (_R, _NP), 1).astype(jnp.float32)

    sc0 = sc_in[...]
    sc_init = jnp.where(sc0 > CONF_TH, sc0, NEG)
    sc_s[...] = sc_init

    macc0 = sc_init[:, 0:_CK]
    for c in range(1, _NCK):
        macc0 = jnp.maximum(macc0, sc_init[:, c * _CK : (c + 1) * _CK])

    def bcast(a):
        return jnp.broadcast_to(a, (_R, _CK))

    def step(t, macc):
        m = jnp.max(macc, axis=1, keepdims=True)
        mb = bcast(m)
        iacc = None
        for c in range(_NCK):
            s = slice(c * _CK, (c + 1) * _CK)
            cand = jnp.where(sc_s[:, s] == mb, io_s[:, s], _BIGI)
            iacc = jnp.minimum(iacc, cand) if c else cand
        idxb = bcast(jnp.min(iacc, axis=1, keepdims=True))
        px0 = py0 = px1 = py1 = None
        for c in range(_NCK):
            s = slice(c * _CK, (c + 1) * _CK)
            oh = io_s[:, s] == idxb
            if c == 0:
                px0 = jnp.where(oh, xs_s[:, s], 0.0)
                py0 = jnp.where(oh, ys_s[:, s], 0.0)
                px1 = jnp.where(oh, xe_s[:, s], 0.0)
                py1 = jnp.where(oh, ye_s[:, s], 0.0)
            else:
                px0 = jnp.where(oh, xs_s[:, s], px0)
                py0 = jnp.where(oh, ys_s[:, s], py0)
                px1 = jnp.where(oh, xe_s[:, s], px1)
                py1 = jnp.where(oh, ye_s[:, s], py1)
        bx0 = jnp.sum(px0, axis=1, keepdims=True)
        by0 = jnp.sum(py0, axis=1, keepdims=True)
        bx1 = jnp.sum(px1, axis=1, keepdims=True)
        by1 = jnp.sum(py1, axis=1, keepdims=True)
        area1 = jnp.maximum(bx1 - bx0, 0.0) * jnp.maximum(by1 - by0, 0.0)
        valid = m > CONF_TH
        bx0b = bcast(bx0)
        by0b = bcast(by0)
        bx1b = bcast(bx1)
        by1b = bcast(by1)
        a1b = bcast(area1)
        macc_n = None
        for c in range(_NCK):
            s = slice(c * _CK, (c + 1) * _CK)
            iw = jnp.maximum(
                jnp.minimum(bx1b, xe_s[:, s]) - jnp.maximum(bx0b, xs_s[:, s]), 0.0
            )
            ih = jnp.maximum(
                jnp.minimum(by1b, ye_s[:, s]) - jnp.maximum(by0b, ys_s[:, s]), 0.0
            )
            inter = iw * ih
            union = a1b + a2_s[:, s] - inter
            iou = inter / jnp.maximum(union, 1e-8)
            sc_new = jnp.where(iou >= IOU_TH, NEG, sc_s[:, s])
            sc_s[:, s] = sc_new
            macc_n = sc_new if c == 0 else jnp.maximum(macc_n, sc_new)
        vrow = valid[:, 0]
        out_ref[0, t, :] = jnp.where(vrow, m[:, 0], 0.0)
        out_ref[1, t, :] = jnp.where(vrow, bx0[:, 0], 0.0)
        out_ref[2, t, :] = jnp.where(vrow, by0[:, 0], 0.0)
        out_ref[3, t, :] = jnp.where(vrow, bx1[:, 0], 0.0)
        out_ref[4, t, :] = jnp.where(vrow, by1[:, 0], 0.0)
        return macc_n

    jax.lax.fori_loop(0, NMS_MAX, step, macc0)


def _topk_body(sc_ref, f_ref, out_ref):
    iota = jax.lax.broadcasted_iota(jnp.int32, (_B, _FLATP), 1)
    cls = (iota // NMS_MAX + 1).astype(jnp.float32)
    f0 = f_ref[0]
    f1 = f_ref[1]
    f2 = f_ref[2]
    f3 = f_ref[3]

    def pick(onehot, a):
        return jnp.sum(jnp.where(onehot, a, 0.0), axis=1)

    def step(k, sc):
        m = jnp.max(sc, axis=1, keepdims=True)
        idx = jnp.min(jnp.where(sc == m, iota, _FLATP), axis=1, keepdims=True)
        onehot = iota == idx
        out_ref[0, k, :] = pick(onehot, cls)
        out_ref[1, k, :] = m[:, 0]
        out_ref[2, k, :] = pick(onehot, f0)
        out_ref[3, k, :] = pick(onehot, f1)
        out_ref[4, k, :] = pick(onehot, f2)
        out_ref[5, k, :] = pick(onehot, f3)
        return jnp.where(onehot, NEG, sc)

    jax.lax.fori_loop(0, NUM_PRED, step, sc_ref[...])


@jax.jit
def kernel(y_pred):
    yt = jnp.transpose(y_pred, (2, 0, 1))
    yt = jnp.pad(yt, ((0, 0), (0, 0), (0, _NP - _N)))
    sc_in = jnp.reshape(yt[1:N_CLASSES].transpose(1, 0, 2), (_R, _NP))
    box_in = yt[N_CLASSES:]

    sel = pl.pallas_call(
        _nms_body,
        out_shape=jax.ShapeDtypeStruct((5, NMS_MAX, _R), jnp.float32),
        scratch_shapes=[pltpu.VMEM((_R, _NP), jnp.float32)] * 7,
    )(sc_in, box_in)

    flat = jnp.reshape(
        jnp.transpose(jnp.reshape(sel, (5, NMS_MAX, _B, _C)), (0, 2, 3, 1)),
        (5, _B, _FLAT),
    )
    sc_flat = jnp.pad(flat[0], ((0, 0), (0, _FLATP - _FLAT)), constant_values=NEG)
    f_flat = jnp.pad(flat[1:], ((0, 0), (0, 0), (0, _FLATP - _FLAT)))

    top = pl.pallas_call(
        _topk_body,
        out_shape=jax.ShapeDtypeStruct((6, NUM_PRED, _B), jnp.float32),
    )(sc_flat, f_flat)

    return jnp.transpose(top, (2, 1, 0))
